# baseline (speedup 1.0000x reference)
import functools

import jax
import jax.numpy as jnp
from jax import lax
from jax.experimental import pallas as pl
from jax.experimental.pallas import tpu as pltpu
from jax.experimental.pallas import tpu_sc as plsc

NC = 2
NS = 16
NW = NC * NS


def _body(idx_hbm, pe_hbm, out_hbm, idx_v, *scratch, b_per_w, chunk, ring):
    bufs = scratch[:ring]
    gsems = scratch[ring:2 * ring]
    wsems = scratch[2 * ring:3 * ring]
    wid = lax.axis_index("s") * NC + lax.axis_index("c")
    base = wid * b_per_w
    n_chunks = b_per_w // chunk
    pltpu.sync_copy(idx_hbm.at[wid], idx_v)

    def start_read(b, g):
        off = (base + g * chunk) % 8192
        pltpu.make_async_copy(
            pe_hbm.at[pl.ds(off, chunk)], bufs[b], gsems[b]
        ).start()

    def wait_read(b, g):
        off = (base + g * chunk) % 8192
        pltpu.make_async_copy(
            pe_hbm.at[pl.ds(off, chunk)], bufs[b], gsems[b]
        ).wait()

    def start_write(b, g):
        pltpu.make_async_copy(
            bufs[b], out_hbm.at[idx_v.at[g]], wsems[b]
        ).start()

    def wait_write(b, g):
        pltpu.make_async_copy(
            bufs[b], out_hbm.at[idx_v.at[g]], wsems[b]
        ).wait()

    for b in range(ring):
        start_read(b, b)

    def step(i, carry):
        g0 = i * ring
        for b in range(ring):
            wait_read(b, g0 + b)
            start_write(b, g0 + b)
        for b in range(ring):
            nxt = g0 + ring + b

            @pl.when(nxt < n_chunks)
            def _():
                wait_write(b, g0 + b)
                start_read(b, nxt)

        return carry

    lax.fori_loop(0, n_chunks // ring, step, 0)
    for b in range(ring):
        wait_write(b, n_chunks - ring + b)


def kernel(token_positions, pe):
    batch, seq = token_positions.shape
    n = batch * seq
    v, d = pe.shape
    b_per_w = n // NW
    chunk = 8
    ring = 8

    n_chunks = b_per_w // chunk
    flat_idx = token_positions.reshape(NW, n_chunks, chunk)

    mesh = plsc.VectorSubcoreMesh(core_axis_name="c", subcore_axis_name="s")
    k = pl.kernel(
        functools.partial(_body, b_per_w=b_per_w, chunk=chunk, ring=ring),
        out_type=jax.ShapeDtypeStruct((n, d), jnp.float32),
        mesh=mesh,
        scratch_types=(
            [pltpu.VMEM((n_chunks, chunk), jnp.int32)]
            + [pltpu.VMEM((chunk, d), jnp.float32) for _ in range(ring)]
            + [pltpu.SemaphoreType.DMA for _ in range(2 * ring)]
        ),
    )
    out = k(flat_idx, pe)
    return out.reshape(batch, seq, d)


# speedup vs baseline: 1.0574x; 1.0574x over previous
import functools

import jax
import jax.numpy as jnp
from jax import lax
from jax.experimental import pallas as pl
from jax.experimental.pallas import tpu as pltpu
from jax.experimental.pallas import tpu_sc as plsc

NC = 2
NS = 16
NW = NC * NS


def _gather_body(idx_hbm, pe_hbm, out_hbm, idx_v, *scratch, b_per_w, chunk, ring):
    nslots = ring // 2
    bufs = scratch[:ring]
    shr = scratch[ring]
    gsems = scratch[ring + 1:ring + 1 + ring]
    wsems = scratch[ring + 1 + ring:ring + 1 + ring + nslots]
    bsems = scratch[ring + 1 + ring + nslots:]
    sid = lax.axis_index("s")
    wid = sid * NC + lax.axis_index("c")
    base = wid * b_per_w
    pltpu.sync_copy(idx_hbm.at[pl.ds(base, b_per_w)], idx_v)
    n_chunks = b_per_w // chunk

    def start_gather(b, g):
        pltpu.make_async_copy(
            pe_hbm.at[idx_v.at[pl.ds(g * chunk, chunk)]], bufs[b], gsems[b]
        ).start()

    def wait_gather(b, g):
        pltpu.make_async_copy(
            pe_hbm.at[idx_v.at[pl.ds(g * chunk, chunk)]], bufs[b], gsems[b]
        ).wait()

    def start_write(b, g):
        pltpu.make_async_copy(
            bufs[b], out_hbm.at[pl.ds(base + g * chunk, chunk)], wsems[b // 2]
        ).start()

    def wait_write(b, g):
        pltpu.make_async_copy(
            bufs[b], out_hbm.at[pl.ds(base + g * chunk, chunk)], wsems[b // 2]
        ).wait()

    def start_bounce(s, g):
        pltpu.make_async_copy(
            shr.at[sid, s], out_hbm.at[pl.ds(base + g * chunk, chunk)], bsems[s]
        ).start()

    def wait_bounce(s, g):
        pltpu.make_async_copy(
            shr.at[sid, s], out_hbm.at[pl.ds(base + g * chunk, chunk)], bsems[s]
        ).wait()

    for b in range(ring):
        start_gather(b, b)

    def step(i, carry):
        g0 = i * ring
        for b in range(ring):
            g = g0 + b
            wait_gather(b, g)
            if b % 2 == 0:
                start_write(b, g)
            else:
                s = (b - 1) // 2

                @pl.when(i > 0)
                def _():
                    wait_bounce(s, g - ring)

                pltpu.sync_copy(bufs[b], shr.at[sid, s])
                start_bounce(s, g)
        for b in range(ring):
            nxt = g0 + ring + b

            @pl.when(nxt < n_chunks)
            def _():
                if b % 2 == 0:
                    wait_write(b, g0 + b)
                start_gather(b, nxt)

        return carry

    lax.fori_loop(0, n_chunks // ring, step, 0)
    for b in range(0, ring, 2):
        wait_write(b, n_chunks - ring + b)
    for s in range(nslots):
        wait_bounce(s, n_chunks - ring + 2 * s + 1)


def kernel(token_positions, pe):
    batch, seq = token_positions.shape
    n = batch * seq
    v, d = pe.shape
    b_per_w = n // NW
    chunk = 8
    ring = 8
    nslots = ring // 2

    flat_idx = token_positions.reshape(n)

    mesh = plsc.VectorSubcoreMesh(core_axis_name="c", subcore_axis_name="s")
    k = pl.kernel(
        functools.partial(_gather_body, b_per_w=b_per_w, chunk=chunk, ring=ring),
        out_type=jax.ShapeDtypeStruct((n, d), jnp.float32),
        mesh=mesh,
        scratch_types=(
            [pltpu.VMEM((b_per_w,), jnp.int32)]
            + [pltpu.VMEM((chunk, d), jnp.float32) for _ in range(ring)]
            + [pltpu.VMEM_SHARED((NS, nslots, chunk, d), jnp.float32)]
            + [pltpu.SemaphoreType.DMA for _ in range(ring + 2 * nslots)]
        ),
    )
    out = k(flat_idx, pe)
    return out.reshape(batch, seq, d)
